# baseline (speedup 1.0000x reference)
import jax
import jax.numpy as jnp
from jax import lax
from jax.experimental import pallas as pl
from jax.experimental.pallas import tpu as pltpu
from jax.experimental.pallas import tpu_sc as plsc

_N = 10000
_E = 320000
_D = 128
_H = _D // 2
_L = 3
_BNS = 1.0 / (1.0 + 1e-5) ** 0.5

_NC = 2
_NS = 16
_NW = _NC * _NS
_EPT = _E // _NW
_C = 80
_NCH = _EPT // _C
_STRIPE = 624
_TAIL = _N - _NS * _STRIPE


def _stripe_copy(src, dst, s):
    pltpu.sync_copy(src.at[pl.ds(s * _STRIPE, _STRIPE)],
                    dst.at[pl.ds(s * _STRIPE, _STRIPE)])

    @pl.when(s == 0)
    def _():
        pltpu.sync_copy(src.at[pl.ds(_NS * _STRIPE, _TAIL)],
                        dst.at[pl.ds(_NS * _STRIPE, _TAIL)])


_B = 8
_G = 4
_NCHM = (_NCH // _B) * _B


def _edge_pass(h2_hbm, srci, dstv, rows, gsem, ssem, accum):
    for b in range(_G):
        pltpu.async_copy(h2_hbm.at[srci.at[b]], rows[b], gsem[b])

    def step(k, b):
        bn = (b + _G) % _B

        @pl.when(k >= _G)
        def _():
            pltpu.make_async_copy(rows[bn], accum.at[dstv.at[k - _G]],
                                  ssem[bn]).wait()

        pltpu.async_copy(h2_hbm.at[srci.at[k + _G]], rows[bn], gsem[bn])
        pltpu.make_async_copy(h2_hbm.at[srci.at[k]], rows[b], gsem[b]).wait()
        pltpu.async_copy(rows[b], accum.at[dstv.at[k]], ssem[b], add=True)

    def outer(t, carry):
        for b in range(_B):
            step(t * _B + b, b)
        return carry

    lax.fori_loop(0, _NCHM // _B, outer, 0)
    for k in range(_NCHM, _NCH):
        b = k % _B
        bn = (b + _G) % _B
        pltpu.make_async_copy(rows[bn], accum.at[dstv.at[k - _G]],
                              ssem[bn]).wait()
        if k + _G < _NCH:
            pltpu.async_copy(h2_hbm.at[srci.at[k + _G]], rows[bn], gsem[bn])
        pltpu.make_async_copy(h2_hbm.at[srci.at[k]], rows[b], gsem[b]).wait()
        pltpu.async_copy(rows[b], accum.at[dstv.at[k]], ssem[b], add=True)
    for k in range(_NCH - _G, _NCH):
        b = k % _B
        pltpu.make_async_copy(rows[b], accum.at[dstv.at[k]], ssem[b]).wait()


def _agg_body(h2_hbm, src_hbm, dst_hbm, zero_hbm,
              p0a_hbm, p0b_hbm, p1a_hbm, p1b_hbm,
              srcv, src_a, src_b, dstv, rows, gsem, ssem, accum):
    c = lax.axis_index("c")
    s = lax.axis_index("s")
    wid = c * _NS + s
    pltpu.sync_copy(src_hbm.at[wid], srcv)
    pltpu.sync_copy(dst_hbm.at[wid], dstv)
    _stripe_copy(zero_hbm, accum, s)

    def tloop(j, carry):
        for k in range(_C // 16):
            v = srcv[j, pl.ds(16 * k, 16)]
            v2 = v + v
            src_a[j, pl.ds(16 * k, 16)] = v2
            src_b[j, pl.ds(16 * k, 16)] = v2 + 1
        return carry

    lax.fori_loop(0, _NCH, tloop, 0)
    plsc.subcore_barrier()

    _edge_pass(h2_hbm, src_a, dstv, rows, gsem, ssem, accum)
    plsc.subcore_barrier()

    @pl.when(c == 0)
    def _():
        _stripe_copy(accum, p0a_hbm, s)

    @pl.when(c == 1)
    def _():
        _stripe_copy(accum, p1a_hbm, s)

    _stripe_copy(zero_hbm, accum, s)
    plsc.subcore_barrier()

    _edge_pass(h2_hbm, src_b, dstv, rows, gsem, ssem, accum)
    plsc.subcore_barrier()

    @pl.when(c == 0)
    def _():
        _stripe_copy(accum, p0b_hbm, s)

    @pl.when(c == 1)
    def _():
        _stripe_copy(accum, p1b_hbm, s)


_half = jax.ShapeDtypeStruct((_N, _H), jnp.float32)

_agg = pl.kernel(
    _agg_body,
    out_type=(_half, _half, _half, _half),
    mesh=plsc.VectorSubcoreMesh(core_axis_name="c", subcore_axis_name="s"),
    compiler_params=pltpu.CompilerParams(use_tc_tiling_on_sc=False),
    scratch_types=[
        pltpu.VMEM((_NCH, _C), jnp.int32),
        pltpu.VMEM((_NCH, _C), jnp.int32),
        pltpu.VMEM((_NCH, _C), jnp.int32),
        pltpu.VMEM((_NCH, _C), jnp.int32),
        [pltpu.VMEM((_C, _H), jnp.float32) for _ in range(_B)],
        [pltpu.SemaphoreType.DMA for _ in range(_B)],
        [pltpu.SemaphoreType.DMA for _ in range(_B)],
        pltpu.VMEM_SHARED((_N, _H), jnp.float32),
    ],
    name="gin_sc_scatter_add",
)

_R = 2000


def _enc_block(x_ref, w_ref, b_ref, o_ref):
    o_ref[...] = jnp.maximum(
        jnp.dot(x_ref[...], w_ref[...], preferred_element_type=jnp.float32)
        + b_ref[...], 0.0)


def _encoder(x, w0t, b0):
    return pl.pallas_call(
        _enc_block,
        grid=(_N // _R,),
        in_specs=[pl.BlockSpec((_R, _D), lambda i: (i, 0)),
                  pl.BlockSpec((_D, _D), lambda i: (0, 0)),
                  pl.BlockSpec((1, _D), lambda i: (0, 0))],
        out_specs=pl.BlockSpec((_R, _D), lambda i: (i, 0)),
        out_shape=jax.ShapeDtypeStruct((_N, _D), jnp.float32),
        name="gin_encoder",
    )(x, w0t, b0)


def _mlp_block(eps_ref, h_ref, p0a_ref, p0b_ref, p1a_ref, p1b_ref,
               w1_ref, t1_ref, w2_ref, t2_ref, so_ref, to_ref, o_ref):
    agg = jnp.concatenate([p0a_ref[...] + p1a_ref[...],
                           p0b_ref[...] + p1b_ref[...]], axis=1)
    z = h_ref[...] * eps_ref[0] + agg
    z = jnp.dot(z, w1_ref[...], preferred_element_type=jnp.float32) + t1_ref[...]
    z = jnp.maximum(z, 0.0)
    z = jnp.dot(z, w2_ref[...], preferred_element_type=jnp.float32) + t2_ref[...]
    z = jnp.maximum(z, 0.0)
    z = jnp.maximum(z * so_ref[...] + to_ref[...], 0.0)
    o_ref[...] = z


def _mlp(eps1, h, p0a, p0b, p1a, p1b, w1f, t1, w2f, t2, so, to):
    vec = pl.BlockSpec((1, _D), lambda i: (0, 0))
    mat = pl.BlockSpec((_D, _D), lambda i: (0, 0))
    blk = pl.BlockSpec((_R, _D), lambda i: (i, 0))
    hlf = pl.BlockSpec((_R, _H), lambda i: (i, 0))
    return pl.pallas_call(
        _mlp_block,
        grid=(_N // _R,),
        in_specs=[pl.BlockSpec(memory_space=pltpu.SMEM),
                  blk, hlf, hlf, hlf, hlf, mat, vec, mat, vec, vec, vec],
        out_specs=blk,
        out_shape=jax.ShapeDtypeStruct((_N, _D), jnp.float32),
        name="gin_mlp",
    )(eps1, h, p0a, p0b, p1a, p1b, w1f, t1, w2f, t2, so, to)


def kernel(x, edge_index, W0, b0, W1, b1, g1, be1, W2, b2, g2, be2, go, bo, eps_v):
    src = edge_index[0].reshape(_NW, _NCH, _C)
    dst = edge_index[1].reshape(_NW, _NCH, _C)
    zeros = jnp.zeros((_N, _H), jnp.float32)
    s1 = _BNS * g1
    w1f = jnp.swapaxes(W1, 1, 2) * s1[:, None, :]
    t1 = (b1 * s1 + be1).reshape(_L, 1, _D)
    s2 = _BNS * g2
    w2f = jnp.swapaxes(W2, 1, 2) * s2[:, None, :]
    t2 = (b2 * s2 + be2).reshape(_L, 1, _D)
    so = (_BNS * go).reshape(_L, 1, _D)
    to = bo.reshape(_L, 1, _D)

    h = _encoder(x, W0.T, b0.reshape(1, _D))
    for i in range(_L):
        h2 = h.reshape(2 * _N, _H)
        p0a, p0b, p1a, p1b = _agg(h2, src, dst, zeros)
        eps1 = (1.0 + eps_v[i]).reshape(1)
        h = _mlp(eps1, h, p0a, p0b, p1a, p1b,
                 w1f[i], t1[i], w2f[i], t2[i], so[i], to[i])
    return h


# speedup vs baseline: 1.0209x; 1.0209x over previous
import jax
import jax.numpy as jnp
from jax import lax
from jax.experimental import pallas as pl
from jax.experimental.pallas import tpu as pltpu
from jax.experimental.pallas import tpu_sc as plsc

_N = 10000
_E = 320000
_D = 128
_H = _D // 2
_L = 3
_BNS = 1.0 / (1.0 + 1e-5) ** 0.5

_NC = 2
_NS = 16
_NW = _NC * _NS
_EPT = _E // _NW
_C = 80
_NCH = _EPT // _C
_STRIPE = 624
_TAIL = _N - _NS * _STRIPE


def _stripe_copy(src, dst, s):
    pltpu.sync_copy(src.at[pl.ds(s * _STRIPE, _STRIPE)],
                    dst.at[pl.ds(s * _STRIPE, _STRIPE)])

    @pl.when(s == 0)
    def _():
        pltpu.sync_copy(src.at[pl.ds(_NS * _STRIPE, _TAIL)],
                        dst.at[pl.ds(_NS * _STRIPE, _TAIL)])


_B = 8
_G = 4
_NCHM = (_NCH // _B) * _B


def _edge_pass(h2_hbm, srci, dstv, rows, gsem, ssem, accum):
    for b in range(_G):
        pltpu.async_copy(h2_hbm.at[srci.at[b]], rows[b], gsem[b])

    def step(k, b):
        bn = (b + _G) % _B

        @pl.when(k >= _G)
        def _():
            pltpu.make_async_copy(rows[bn], accum.at[dstv.at[k - _G]],
                                  ssem[bn]).wait()

        pltpu.async_copy(h2_hbm.at[srci.at[k + _G]], rows[bn], gsem[bn])
        pltpu.make_async_copy(h2_hbm.at[srci.at[k]], rows[b], gsem[b]).wait()
        pltpu.async_copy(rows[b], accum.at[dstv.at[k]], ssem[b], add=True)

    def outer(t, carry):
        for b in range(_B):
            step(t * _B + b, b)
        return carry

    lax.fori_loop(0, _NCHM // _B, outer, 0)
    for k in range(_NCHM, _NCH):
        b = k % _B
        bn = (b + _G) % _B
        pltpu.make_async_copy(rows[bn], accum.at[dstv.at[k - _G]],
                              ssem[bn]).wait()
        if k + _G < _NCH:
            pltpu.async_copy(h2_hbm.at[srci.at[k + _G]], rows[bn], gsem[bn])
        pltpu.make_async_copy(h2_hbm.at[srci.at[k]], rows[b], gsem[b]).wait()
        pltpu.async_copy(rows[b], accum.at[dstv.at[k]], ssem[b], add=True)
    for k in range(_NCH - _G, _NCH):
        b = k % _B
        pltpu.make_async_copy(rows[b], accum.at[dstv.at[k]], ssem[b]).wait()


def _agg_body(h2_hbm, src_hbm, dst_hbm, zero_hbm,
              p0a_hbm, p0b_hbm, p1a_hbm, p1b_hbm,
              srcv, dstv, rows, gsem, ssem, zsem, accum):
    c = lax.axis_index("c")
    s = lax.axis_index("s")
    wid = c * _NS + s
    pltpu.async_copy(zero_hbm.at[pl.ds(s * _STRIPE, _STRIPE)],
                     accum.at[pl.ds(s * _STRIPE, _STRIPE)], zsem)

    @pl.when(s == 0)
    def _():
        pltpu.async_copy(zero_hbm.at[pl.ds(_NS * _STRIPE, _TAIL)],
                         accum.at[pl.ds(_NS * _STRIPE, _TAIL)], zsem)

    pltpu.sync_copy(src_hbm.at[wid], srcv)
    pltpu.sync_copy(dst_hbm.at[wid], dstv)

    def tloop(j, carry):
        for k in range(_C // 16):
            v = srcv[j, pl.ds(16 * k, 16)]
            srcv[j, pl.ds(16 * k, 16)] = v + v
        return carry

    lax.fori_loop(0, _NCH, tloop, 0)
    pltpu.make_async_copy(zero_hbm.at[pl.ds(s * _STRIPE, _STRIPE)],
                          accum.at[pl.ds(s * _STRIPE, _STRIPE)], zsem).wait()

    @pl.when(s == 0)
    def _():
        pltpu.make_async_copy(zero_hbm.at[pl.ds(_NS * _STRIPE, _TAIL)],
                              accum.at[pl.ds(_NS * _STRIPE, _TAIL)],
                              zsem).wait()

    plsc.subcore_barrier()

    _edge_pass(h2_hbm, srcv, dstv, rows, gsem, ssem, accum)
    plsc.subcore_barrier()

    @pl.when(c == 0)
    def _():
        _stripe_copy(accum, p0a_hbm, s)

    @pl.when(c == 1)
    def _():
        _stripe_copy(accum, p1a_hbm, s)

    _stripe_copy(zero_hbm, accum, s)
    plsc.subcore_barrier()

    _edge_pass(h2_hbm.at[pl.ds(1, 2 * _N - 1)], srcv, dstv,
               rows, gsem, ssem, accum)
    plsc.subcore_barrier()

    @pl.when(c == 0)
    def _():
        _stripe_copy(accum, p0b_hbm, s)

    @pl.when(c == 1)
    def _():
        _stripe_copy(accum, p1b_hbm, s)


_half = jax.ShapeDtypeStruct((_N, _H), jnp.float32)

_agg = pl.kernel(
    _agg_body,
    out_type=(_half, _half, _half, _half),
    mesh=plsc.VectorSubcoreMesh(core_axis_name="c", subcore_axis_name="s"),
    compiler_params=pltpu.CompilerParams(use_tc_tiling_on_sc=False),
    scratch_types=[
        pltpu.VMEM((_NCH, _C), jnp.int32),
        pltpu.VMEM((_NCH, _C), jnp.int32),
        [pltpu.VMEM((_C, _H), jnp.float32) for _ in range(_B)],
        [pltpu.SemaphoreType.DMA for _ in range(_B)],
        [pltpu.SemaphoreType.DMA for _ in range(_B)],
        pltpu.SemaphoreType.DMA,
        pltpu.VMEM_SHARED((_N, _H), jnp.float32),
    ],
    name="gin_sc_scatter_add",
)

_R = 2000


def _enc_block(x_ref, w_ref, b_ref, o_ref):
    o_ref[...] = jnp.maximum(
        jnp.dot(x_ref[...], w_ref[...], preferred_element_type=jnp.float32)
        + b_ref[...], 0.0)


def _encoder(x, w0t, b0):
    return pl.pallas_call(
        _enc_block,
        grid=(_N // _R,),
        in_specs=[pl.BlockSpec((_R, _D), lambda i: (i, 0)),
                  pl.BlockSpec((_D, _D), lambda i: (0, 0)),
                  pl.BlockSpec((1, _D), lambda i: (0, 0))],
        out_specs=pl.BlockSpec((_R, _D), lambda i: (i, 0)),
        out_shape=jax.ShapeDtypeStruct((_N, _D), jnp.float32),
        name="gin_encoder",
    )(x, w0t, b0)


def _mlp_block(eps_ref, h_ref, p0a_ref, p0b_ref, p1a_ref, p1b_ref,
               w1_ref, t1_ref, w2_ref, t2_ref, so_ref, to_ref, o_ref):
    agg = jnp.concatenate([p0a_ref[...] + p1a_ref[...],
                           p0b_ref[...] + p1b_ref[...]], axis=1)
    z = h_ref[...] * eps_ref[0] + agg
    z = jnp.dot(z, w1_ref[...], preferred_element_type=jnp.float32) + t1_ref[...]
    z = jnp.maximum(z, 0.0)
    z = jnp.dot(z, w2_ref[...], preferred_element_type=jnp.float32) + t2_ref[...]
    z = jnp.maximum(z, 0.0)
    z = jnp.maximum(z * so_ref[...] + to_ref[...], 0.0)
    o_ref[...] = z


def _mlp(eps1, h, p0a, p0b, p1a, p1b, w1f, t1, w2f, t2, so, to):
    vec = pl.BlockSpec((1, _D), lambda i: (0, 0))
    mat = pl.BlockSpec((_D, _D), lambda i: (0, 0))
    blk = pl.BlockSpec((_R, _D), lambda i: (i, 0))
    hlf = pl.BlockSpec((_R, _H), lambda i: (i, 0))
    return pl.pallas_call(
        _mlp_block,
        grid=(_N // _R,),
        in_specs=[pl.BlockSpec(memory_space=pltpu.SMEM),
                  blk, hlf, hlf, hlf, hlf, mat, vec, mat, vec, vec, vec],
        out_specs=blk,
        out_shape=jax.ShapeDtypeStruct((_N, _D), jnp.float32),
        name="gin_mlp",
    )(eps1, h, p0a, p0b, p1a, p1b, w1f, t1, w2f, t2, so, to)


def kernel(x, edge_index, W0, b0, W1, b1, g1, be1, W2, b2, g2, be2, go, bo, eps_v):
    src = edge_index[0].reshape(_NW, _NCH, _C)
    dst = edge_index[1].reshape(_NW, _NCH, _C)
    zeros = jnp.zeros((_N, _H), jnp.float32)
    s1 = _BNS * g1
    w1f = jnp.swapaxes(W1, 1, 2) * s1[:, None, :]
    t1 = (b1 * s1 + be1).reshape(_L, 1, _D)
    s2 = _BNS * g2
    w2f = jnp.swapaxes(W2, 1, 2) * s2[:, None, :]
    t2 = (b2 * s2 + be2).reshape(_L, 1, _D)
    so = (_BNS * go).reshape(_L, 1, _D)
    to = bo.reshape(_L, 1, _D)

    h = _encoder(x, W0.T, b0.reshape(1, _D))
    for i in range(_L):
        h2 = h.reshape(2 * _N, _H)
        p0a, p0b, p1a, p1b = _agg(h2, src, dst, zeros)
        eps1 = (1.0 + eps_v[i]).reshape(1)
        h = _mlp(eps1, h, p0a, p0b, p1a, p1b,
                 w1f[i], t1[i], w2f[i], t2[i], so[i], to[i])
    return h


# speedup vs baseline: 1.0220x; 1.0011x over previous
import jax
import jax.numpy as jnp
from jax import lax
from jax.experimental import pallas as pl
from jax.experimental.pallas import tpu as pltpu
from jax.experimental.pallas import tpu_sc as plsc

_N = 10000
_E = 320000
_D = 128
_H = _D // 2
_L = 3
_BNS = 1.0 / (1.0 + 1e-5) ** 0.5

_NC = 2
_NS = 16
_NW = _NC * _NS
_EPT = _E // _NW
_C = 80
_NCH = _EPT // _C
_STRIPE = 624
_TAIL = _N - _NS * _STRIPE


def _stripe_copy(src, dst, s):
    pltpu.sync_copy(src.at[pl.ds(s * _STRIPE, _STRIPE)],
                    dst.at[pl.ds(s * _STRIPE, _STRIPE)])

    @pl.when(s == 0)
    def _():
        pltpu.sync_copy(src.at[pl.ds(_NS * _STRIPE, _TAIL)],
                        dst.at[pl.ds(_NS * _STRIPE, _TAIL)])


_B = 10
_G = 5
_NCHM = (_NCH // _B) * _B


def _edge_pass(h2_hbm, srci, dstv, rows, gsem, ssem, accum):
    for b in range(_G):
        pltpu.async_copy(h2_hbm.at[srci.at[b]], rows[b], gsem[b])

    def step(k, b):
        bn = (b + _G) % _B

        @pl.when(k >= _G)
        def _():
            pltpu.make_async_copy(rows[bn], accum.at[dstv.at[k - _G]],
                                  ssem[bn]).wait()

        pltpu.async_copy(h2_hbm.at[srci.at[k + _G]], rows[bn], gsem[bn])
        pltpu.make_async_copy(h2_hbm.at[srci.at[k]], rows[b], gsem[b]).wait()
        pltpu.async_copy(rows[b], accum.at[dstv.at[k]], ssem[b], add=True)

    def outer(t, carry):
        for b in range(_B):
            step(t * _B + b, b)
        return carry

    lax.fori_loop(0, _NCHM // _B, outer, 0)
    for k in range(_NCHM, _NCH):
        b = k % _B
        bn = (b + _G) % _B
        pltpu.make_async_copy(rows[bn], accum.at[dstv.at[k - _G]],
                              ssem[bn]).wait()
        if k + _G < _NCH:
            pltpu.async_copy(h2_hbm.at[srci.at[k + _G]], rows[bn], gsem[bn])
        pltpu.make_async_copy(h2_hbm.at[srci.at[k]], rows[b], gsem[b]).wait()
        pltpu.async_copy(rows[b], accum.at[dstv.at[k]], ssem[b], add=True)
    for k in range(_NCH - _G, _NCH):
        b = k % _B
        pltpu.make_async_copy(rows[b], accum.at[dstv.at[k]], ssem[b]).wait()


def _agg_body(h2_hbm, src_hbm, dst_hbm, zero_hbm,
              p0a_hbm, p0b_hbm, p1a_hbm, p1b_hbm,
              srcv, dstv, rows, gsem, ssem, zsem, accum):
    c = lax.axis_index("c")
    s = lax.axis_index("s")
    wid = c * _NS + s
    pltpu.async_copy(zero_hbm.at[pl.ds(s * _STRIPE, _STRIPE)],
                     accum.at[pl.ds(s * _STRIPE, _STRIPE)], zsem)

    @pl.when(s == 0)
    def _():
        pltpu.async_copy(zero_hbm.at[pl.ds(_NS * _STRIPE, _TAIL)],
                         accum.at[pl.ds(_NS * _STRIPE, _TAIL)], zsem)

    pltpu.sync_copy(src_hbm.at[wid], srcv)
    pltpu.sync_copy(dst_hbm.at[wid], dstv)

    def tloop(j, carry):
        for k in range(_C // 16):
            v = srcv[j, pl.ds(16 * k, 16)]
            srcv[j, pl.ds(16 * k, 16)] = v + v
        return carry

    lax.fori_loop(0, _NCH, tloop, 0)
    pltpu.make_async_copy(zero_hbm.at[pl.ds(s * _STRIPE, _STRIPE)],
                          accum.at[pl.ds(s * _STRIPE, _STRIPE)], zsem).wait()

    @pl.when(s == 0)
    def _():
        pltpu.make_async_copy(zero_hbm.at[pl.ds(_NS * _STRIPE, _TAIL)],
                              accum.at[pl.ds(_NS * _STRIPE, _TAIL)],
                              zsem).wait()

    plsc.subcore_barrier()

    _edge_pass(h2_hbm, srcv, dstv, rows, gsem, ssem, accum)
    plsc.subcore_barrier()

    @pl.when(c == 0)
    def _():
        _stripe_copy(accum, p0a_hbm, s)

    @pl.when(c == 1)
    def _():
        _stripe_copy(accum, p1a_hbm, s)

    _stripe_copy(zero_hbm, accum, s)
    plsc.subcore_barrier()

    _edge_pass(h2_hbm.at[pl.ds(1, 2 * _N - 1)], srcv, dstv,
               rows, gsem, ssem, accum)
    plsc.subcore_barrier()

    @pl.when(c == 0)
    def _():
        _stripe_copy(accum, p0b_hbm, s)

    @pl.when(c == 1)
    def _():
        _stripe_copy(accum, p1b_hbm, s)


_half = jax.ShapeDtypeStruct((_N, _H), jnp.float32)

_agg = pl.kernel(
    _agg_body,
    out_type=(_half, _half, _half, _half),
    mesh=plsc.VectorSubcoreMesh(core_axis_name="c", subcore_axis_name="s"),
    compiler_params=pltpu.CompilerParams(use_tc_tiling_on_sc=False),
    scratch_types=[
        pltpu.VMEM((_NCH, _C), jnp.int32),
        pltpu.VMEM((_NCH, _C), jnp.int32),
        [pltpu.VMEM((_C, _H), jnp.float32) for _ in range(_B)],
        [pltpu.SemaphoreType.DMA for _ in range(_B)],
        [pltpu.SemaphoreType.DMA for _ in range(_B)],
        pltpu.SemaphoreType.DMA,
        pltpu.VMEM_SHARED((_N, _H), jnp.float32),
    ],
    name="gin_sc_scatter_add",
)

_R = 2000


def _enc_block(x_ref, w_ref, b_ref, o_ref):
    o_ref[...] = jnp.maximum(
        jnp.dot(x_ref[...], w_ref[...], preferred_element_type=jnp.float32)
        + b_ref[...], 0.0)


def _encoder(x, w0t, b0):
    return pl.pallas_call(
        _enc_block,
        grid=(_N // _R,),
        in_specs=[pl.BlockSpec((_R, _D), lambda i: (i, 0)),
                  pl.BlockSpec((_D, _D), lambda i: (0, 0)),
                  pl.BlockSpec((1, _D), lambda i: (0, 0))],
        out_specs=pl.BlockSpec((_R, _D), lambda i: (i, 0)),
        out_shape=jax.ShapeDtypeStruct((_N, _D), jnp.float32),
        name="gin_encoder",
    )(x, w0t, b0)


def _mlp_block(eps_ref, h_ref, p0a_ref, p0b_ref, p1a_ref, p1b_ref,
               w1_ref, t1_ref, w2_ref, t2_ref, so_ref, to_ref, o_ref):
    agg = jnp.concatenate([p0a_ref[...] + p1a_ref[...],
                           p0b_ref[...] + p1b_ref[...]], axis=1)
    z = h_ref[...] * eps_ref[0] + agg
    z = jnp.dot(z, w1_ref[...], preferred_element_type=jnp.float32) + t1_ref[...]
    z = jnp.maximum(z, 0.0)
    z = jnp.dot(z, w2_ref[...], preferred_element_type=jnp.float32) + t2_ref[...]
    z = jnp.maximum(z, 0.0)
    z = jnp.maximum(z * so_ref[...] + to_ref[...], 0.0)
    o_ref[...] = z


def _mlp(eps1, h, p0a, p0b, p1a, p1b, w1f, t1, w2f, t2, so, to):
    vec = pl.BlockSpec((1, _D), lambda i: (0, 0))
    mat = pl.BlockSpec((_D, _D), lambda i: (0, 0))
    blk = pl.BlockSpec((_R, _D), lambda i: (i, 0))
    hlf = pl.BlockSpec((_R, _H), lambda i: (i, 0))
    return pl.pallas_call(
        _mlp_block,
        grid=(_N // _R,),
        in_specs=[pl.BlockSpec(memory_space=pltpu.SMEM),
                  blk, hlf, hlf, hlf, hlf, mat, vec, mat, vec, vec, vec],
        out_specs=blk,
        out_shape=jax.ShapeDtypeStruct((_N, _D), jnp.float32),
        name="gin_mlp",
    )(eps1, h, p0a, p0b, p1a, p1b, w1f, t1, w2f, t2, so, to)


def kernel(x, edge_index, W0, b0, W1, b1, g1, be1, W2, b2, g2, be2, go, bo, eps_v):
    src = edge_index[0].reshape(_NW, _NCH, _C)
    dst = edge_index[1].reshape(_NW, _NCH, _C)
    zeros = jnp.zeros((_N, _H), jnp.float32)
    s1 = _BNS * g1
    w1f = jnp.swapaxes(W1, 1, 2) * s1[:, None, :]
    t1 = (b1 * s1 + be1).reshape(_L, 1, _D)
    s2 = _BNS * g2
    w2f = jnp.swapaxes(W2, 1, 2) * s2[:, None, :]
    t2 = (b2 * s2 + be2).reshape(_L, 1, _D)
    so = (_BNS * go).reshape(_L, 1, _D)
    to = bo.reshape(_L, 1, _D)

    h = _encoder(x, W0.T, b0.reshape(1, _D))
    for i in range(_L):
        h2 = h.reshape(2 * _N, _H)
        p0a, p0b, p1a, p1b = _agg(h2, src, dst, zeros)
        eps1 = (1.0 + eps_v[i]).reshape(1)
        h = _mlp(eps1, h, p0a, p0b, p1a, p1b,
                 w1f[i], t1[i], w2f[i], t2[i], so[i], to[i])
    return h
